# Initial kernel scaffold; baseline (speedup 1.0000x reference)
import jax
import jax.numpy as jnp
from jax.experimental import pallas as pl


def kernel(n, e, edge_index, eW1, eb1, cb1, g1, bb1, eW2, eb2, cb2, g2, bb2, eW3, eb3, cb3, g3, bb3, eW4, eb4, cb4, g4, bb4, fcW, al, ar, gb, gateW, gateb, fW1, fb1, fW2, fb2, fW3, fb3):
    raise NotImplementedError("write your pallas kernel here")


# speedup vs baseline: 5.8706x; 5.8706x over previous
import functools

import jax
import jax.numpy as jnp
from jax import lax
from jax.experimental import pallas as pl
from jax.experimental.pallas import tpu as pltpu
from jax.experimental.pallas import tpu_sc as plsc

N = 10000
E = 160000
DIN = 16
H = 8
DOUT = 128
HD = H * DOUT

NC = 2
NS = 16
NW = NC * NS
CH = 128
NR = E // CH
RPW = NR // NW
EXTRA = NR - RPW * NW
SUB_N = N // NS

_f32 = jnp.float32


def _wid_trip():
    cid = lax.axis_index("c")
    sid = lax.axis_index("s")
    wid = sid * NC + cid
    trip = jnp.where(wid < EXTRA, RPW + 1, RPW)
    return cid, sid, wid, trip


def _sc_gather_body(table, idx, out, idx_v, rows_v, sem):
    _, _, wid, trip = _wid_trip()

    def body(k, carry):
        r = wid + NW * k
        pltpu.sync_copy(idx.at[r], idx_v)
        pltpu.async_copy(table.at[idx_v], rows_v, sem).wait()
        pltpu.sync_copy(rows_v, out.at[pl.ds(r * CH, CH)])
        return carry

    lax.fori_loop(0, trip, body, 0)


@functools.cache
def _sc_mesh():
    return plsc.VectorSubcoreMesh(core_axis_name="c", subcore_axis_name="s",
                                  num_cores=NC, num_subcores=NS)


@functools.cache
def _gather16_kernel():
    return pl.kernel(
        _sc_gather_body,
        out_type=jax.ShapeDtypeStruct((E, 16), _f32),
        mesh=_sc_mesh(),
        compiler_params=pltpu.CompilerParams(use_tc_tiling_on_sc=False),
        scratch_types=[
            pltpu.VMEM((CH,), jnp.int32),
            pltpu.VMEM((CH, 16), _f32),
            pltpu.SemaphoreType.DMA,
        ],
    )


def _gather16(table, idx2d):
    return _gather16_kernel()(table, idx2d)


def _sc_scatter16_body(vals, idx, zeros, out, acc, idx2_v, v_v):
    cid, sid, wid, trip = _wid_trip()
    pltpu.sync_copy(zeros.at[pl.ds(sid * SUB_N, SUB_N)],
                    acc.at[pl.ds(sid * SUB_N, SUB_N)])
    plsc.subcore_barrier()

    def body(k, carry):
        r = wid + NW * k
        pltpu.sync_copy(idx.at[r], idx2_v.at[k])
        pltpu.sync_copy(vals.at[pl.ds(r * CH, CH)], v_v)
        pltpu.sync_copy(v_v, acc.at[idx2_v.at[k]], add=True)
        return carry

    lax.fori_loop(0, trip, body, 0)
    plsc.subcore_barrier()
    pltpu.sync_copy(acc.at[pl.ds(sid * SUB_N, SUB_N)],
                    out.at[pl.ds(cid * N + sid * SUB_N, SUB_N)])


@functools.cache
def _scatter16_kernel():
    return pl.kernel(
        _sc_scatter16_body,
        out_type=jax.ShapeDtypeStruct((2 * N, 16), _f32),
        mesh=_sc_mesh(),
        compiler_params=pltpu.CompilerParams(use_tc_tiling_on_sc=False),
        scratch_types=[
            pltpu.VMEM_SHARED((N, 16), _f32),
            pltpu.VMEM((RPW + 1, CH), jnp.int32),
            pltpu.VMEM((CH, 16), _f32),
        ],
    )


def _scatter16(vals, idx2d, zeros):
    return _scatter16_kernel()(vals, idx2d, zeros)


def _sc_pass2_body(ee, hs, den, idx, zeros, out,
                   acc, idx2_v, ee_v, hs_v, den_v, mu_v, sem):
    cid, sid, wid, trip = _wid_trip()
    pltpu.sync_copy(zeros.at[pl.ds(sid * SUB_N, SUB_N)],
                    acc.at[pl.ds(sid * SUB_N, SUB_N)])
    plsc.subcore_barrier()

    def chunk(k, carry):
        r = wid + NW * k
        pltpu.sync_copy(idx.at[r], idx2_v.at[k])
        pltpu.sync_copy(ee.at[pl.ds(r * CH, CH)], ee_v)
        pltpu.sync_copy(hs.at[pl.ds(r * CH, CH)], hs_v)
        pltpu.async_copy(den.at[idx2_v.at[k]], den_v, sem).wait()

        def edge(i, c2):
            hs_row = hs_v[i, :]
            al_row = ee_v[i, :] / den_v[i, :]
            for hh in range(H):
                mu_v[i, pl.ds(hh * 16, 16)] = hs_row * al_row[hh]
            return c2

        lax.fori_loop(0, CH, edge, 0)
        pltpu.sync_copy(mu_v, acc.at[idx2_v.at[k]], add=True)
        return carry

    lax.fori_loop(0, trip, chunk, 0)
    plsc.subcore_barrier()
    pltpu.sync_copy(acc.at[pl.ds(sid * SUB_N, SUB_N)],
                    out.at[pl.ds(cid * N + sid * SUB_N, SUB_N)])


@functools.cache
def _pass2_kernel():
    return pl.kernel(
        _sc_pass2_body,
        out_type=jax.ShapeDtypeStruct((2 * N, H * DIN), _f32),
        mesh=_sc_mesh(),
        compiler_params=pltpu.CompilerParams(use_tc_tiling_on_sc=False),
        scratch_types=[
            pltpu.VMEM_SHARED((N, H * DIN), _f32),
            pltpu.VMEM((RPW + 1, CH), jnp.int32),
            pltpu.VMEM((CH, 16), _f32),
            pltpu.VMEM((CH, 16), _f32),
            pltpu.VMEM((CH, 16), _f32),
            pltpu.VMEM((CH, H * DIN), _f32),
            pltpu.SemaphoreType.DMA,
        ],
    )


def _pass2(ee, hs, den, idx2d, zeros):
    return _pass2_kernel()(ee, hs, den, idx2d, zeros)


TE = 4000


def _tc_msg_body(hsrc_ref, e_ref, eW_ref, eb_ref, S_ref, T_ref, out_ref):
    G = jnp.dot(e_ref[...], eW_ref[...],
                preferred_element_type=_f32, precision=lax.Precision.HIGHEST) + eb_ref[...]
    hexp = jnp.dot(hsrc_ref[...], S_ref[...], preferred_element_type=_f32, precision=lax.Precision.HIGHEST)
    out_ref[...] = jnp.dot(G * hexp, T_ref[...], preferred_element_type=_f32, precision=lax.Precision.HIGHEST)


def _msg(hsrc, e, eW, eb, S, T):
    return pl.pallas_call(
        _tc_msg_body,
        grid=(E // TE,),
        in_specs=[
            pl.BlockSpec((TE, 16), lambda i: (i, 0)),
            pl.BlockSpec((TE, 16), lambda i: (i, 0)),
            pl.BlockSpec((16, 256), lambda i: (0, 0)),
            pl.BlockSpec((1, 256), lambda i: (0, 0)),
            pl.BlockSpec((16, 256), lambda i: (0, 0)),
            pl.BlockSpec((256, 16), lambda i: (0, 0)),
        ],
        out_specs=pl.BlockSpec((TE, 16), lambda i: (i, 0)),
        out_shape=jax.ShapeDtypeStruct((E, 16), _f32),
    )(hsrc, e, eW, eb.reshape(1, 256), S, T)


def _bn_core(aggp_ref, cb_ref, g_ref, bb_ref):
    x = aggp_ref[0:N, :] + aggp_ref[N:2 * N, :] + cb_ref[...]
    m = jnp.mean(x, axis=0, keepdims=True)
    xc = x - m
    v = jnp.mean(xc * xc, axis=0, keepdims=True)
    h = g_ref[...] * xc * lax.rsqrt(v + 1e-5) + bb_ref[...]
    return jnp.maximum(h, 0.0)


def _tc_bn_body(aggp_ref, cb_ref, g_ref, bb_ref, out_ref):
    out_ref[...] = _bn_core(aggp_ref, cb_ref, g_ref, bb_ref)


def _bn(aggp, cb, g, bb):
    return pl.pallas_call(
        _tc_bn_body,
        out_shape=jax.ShapeDtypeStruct((N, 16), _f32),
    )(aggp, cb.reshape(1, 16), g.reshape(1, 16), bb.reshape(1, 16))


def _tc_bn4_body(aggp_ref, cb_ref, g_ref, bb_ref, alW_ref, arW_ref,
                 h_ref, elr_ref, rel_ref):
    h = _bn_core(aggp_ref, cb_ref, g_ref, bb_ref)
    h_ref[...] = h
    el = jnp.dot(h, alW_ref[...], preferred_element_type=_f32, precision=lax.Precision.HIGHEST)
    er = jnp.dot(h, arW_ref[...], preferred_element_type=_f32, precision=lax.Precision.HIGHEST)
    elr_ref[...] = jnp.concatenate([el, er], axis=1)
    rel_ref[...] = jnp.concatenate([er, el], axis=1)


def _bn4(aggp, cb, g, bb, alW, arW):
    return pl.pallas_call(
        _tc_bn4_body,
        out_shape=(
            jax.ShapeDtypeStruct((N, 16), _f32),
            jax.ShapeDtypeStruct((N, 16), _f32),
            jax.ShapeDtypeStruct((N, 16), _f32),
        ),
    )(aggp, cb.reshape(1, 16), g.reshape(1, 16), bb.reshape(1, 16), alW, arW)


def _tc_ee_body(es_ref, ed_ref, out_ref):
    x = es_ref[...] + ed_ref[...]
    l = jnp.where(x >= 0, x, 0.2 * x)
    ev = jnp.exp(l)
    lane = lax.broadcasted_iota(jnp.int32, ev.shape, 1)
    out_ref[...] = jnp.where(lane < H, ev, 0.0)


def _ee(esrc, edst):
    return pl.pallas_call(
        _tc_ee_body,
        grid=(E // TE,),
        in_specs=[
            pl.BlockSpec((TE, 16), lambda i: (i, 0)),
            pl.BlockSpec((TE, 16), lambda i: (i, 0)),
        ],
        out_specs=pl.BlockSpec((TE, 16), lambda i: (i, 0)),
        out_shape=jax.ShapeDtypeStruct((E, 16), _f32),
    )(esrc, edst)


def _tc_den_body(dp_ref, out_ref):
    d = dp_ref[0:N, :] + dp_ref[N:2 * N, :]
    lane = lax.broadcasted_iota(jnp.int32, d.shape, 1)
    out_ref[...] = jnp.where(lane < H, d, 1.0)


def _den(denp):
    return pl.pallas_call(
        _tc_den_body,
        out_shape=jax.ShapeDtypeStruct((N, 16), _f32),
    )(denp)


TCH = 2000


def _tc_final_body(Up_ref, Bb_ref, gbf_ref, gW_ref, gb_ref,
                   f1_ref, b1_ref, f2_ref, b2_ref, f3_ref, b3_ref, out_ref,
                   gate_scr):
    Bb = Bb_ref[...]
    gbf = gbf_ref[...]

    def hf_chunk(c):
        Uc = (Up_ref[pl.ds(c * TCH, TCH), :]
              + Up_ref[pl.ds(N + c * TCH, TCH), :])
        x = jnp.dot(Uc, Bb, preferred_element_type=_f32, precision=lax.Precision.HIGHEST) + gbf
        return jnp.where(x > 0, x, jnp.exp(jnp.minimum(x, 0.0)) - 1.0)

    def p1(c, carry):
        hf = hf_chunk(c)
        gate_scr[pl.ds(c * TCH, TCH), :] = (
            jnp.dot(hf, gW_ref[...], preferred_element_type=_f32, precision=lax.Precision.HIGHEST)
            + gb_ref[...])
        return carry

    lax.fori_loop(0, N // TCH, p1, 0)
    gate = gate_scr[...]
    m = jnp.max(gate)
    eg = jnp.exp(gate - m)
    s = jnp.sum(eg)
    gate_scr[...] = eg

    def p2(c, hg):
        hf = hf_chunk(c)
        w = gate_scr[pl.ds(c * TCH, TCH), :]
        return hg + jnp.sum(w * hf, axis=0, keepdims=True)

    hg = lax.fori_loop(0, N // TCH, p2, jnp.zeros((1, HD), _f32)) / s
    z = jnp.maximum(jnp.dot(hg, f1_ref[...], preferred_element_type=_f32, precision=lax.Precision.HIGHEST)
                    + b1_ref[...], 0.0)
    z = jnp.maximum(jnp.dot(z, f2_ref[...], preferred_element_type=_f32, precision=lax.Precision.HIGHEST)
                    + b2_ref[...], 0.0)
    out_ref[...] = jnp.dot(z, f3_ref[...], preferred_element_type=_f32, precision=lax.Precision.HIGHEST) \
        + b3_ref[...]


def _final(Up, Bblk, gbf, gateW, gateb, fW1, fb1, fW2, fb2, fW3, fb3):
    return pl.pallas_call(
        _tc_final_body,
        out_shape=jax.ShapeDtypeStruct((1, 1), _f32),
        scratch_shapes=[pltpu.VMEM((N, 1), _f32)],
    )(Up, Bblk, gbf, gateW, gateb.reshape(1, 1), fW1, fb1.reshape(1, 64),
      fW2, fb2.reshape(1, 32), fW3, fb3.reshape(1, 1))


def kernel(n, e, edge_index,
           eW1, eb1, cb1, g1, bb1,
           eW2, eb2, cb2, g2, bb2,
           eW3, eb3, cb3, g3, bb3,
           eW4, eb4, cb4, g4, bb4,
           fcW, al, ar, gb, gateW, gateb,
           fW1, fb1, fW2, fb2, fW3, fb3):
    src2d = edge_index[0].reshape(NR, CH)
    dst2d = edge_index[1].reshape(NR, CH)
    z16 = jnp.zeros((N, 16), _f32)
    z128 = jnp.zeros((N, H * DIN), _f32)

    fcW3 = fcW.reshape(DIN, H, DOUT)
    alW = jnp.einsum('iho,ho->ih', fcW3, al)
    arW = jnp.einsum('iho,ho->ih', fcW3, ar)
    Bblk = jax.scipy.linalg.block_diag(*[fcW3[:, hh, :] for hh in range(H)])
    eye = jnp.eye(DIN, dtype=_f32)
    S = jnp.kron(eye, jnp.ones((1, 16), _f32))
    T = jnp.tile(eye, (DIN, 1))

    layers = [(eW1, eb1, cb1, g1, bb1), (eW2, eb2, cb2, g2, bb2),
              (eW3, eb3, cb3, g3, bb3), (eW4, eb4, cb4, g4, bb4)]
    h = n
    elr = rel = None
    for li, (eW, eb, cb, g, bb) in enumerate(layers):
        hsrc = _gather16(h, src2d)
        msg = _msg(hsrc, e, eW, eb, S, T)
        aggp = _scatter16(msg, dst2d, z16)
        if li < 3:
            h = _bn(aggp, cb, g, bb)
        else:
            h, elr, rel = _bn4(aggp, cb, g, bb, alW, arW)

    esrc = _gather16(elr, src2d)
    edst = _gather16(rel, dst2d)
    hs4 = _gather16(h, src2d)
    ee16 = _ee(esrc, edst)
    denp = _scatter16(ee16, dst2d, z16)
    den16 = _den(denp)
    Up = _pass2(ee16, hs4, den16, dst2d, z128)
    return _final(Up, Bblk, gb.reshape(1, HD), gateW, gateb,
                  fW1, fb1, fW2, fb2, fW3, fb3)


# speedup vs baseline: 6.1781x; 1.0524x over previous
import functools

import jax
import jax.numpy as jnp
from jax import lax
from jax.experimental import pallas as pl
from jax.experimental.pallas import tpu as pltpu
from jax.experimental.pallas import tpu_sc as plsc

N = 10000
E = 160000
DIN = 16
H = 8
DOUT = 128
HD = H * DOUT

NC = 2
NS = 16
NW = NC * NS
CH = 128
NR = E // CH
RPW = NR // NW
EXTRA = NR - RPW * NW
SUB_N = N // NS

_f32 = jnp.float32


def _wid_trip():
    cid = lax.axis_index("c")
    sid = lax.axis_index("s")
    wid = sid * NC + cid
    trip = jnp.where(wid < EXTRA, RPW + 1, RPW)
    return cid, sid, wid, trip


def _sc_gather_body(table, idx, out, idx_v, rows_v, sem):
    _, _, wid, trip = _wid_trip()

    def body(k, carry):
        r = wid + NW * k
        pltpu.sync_copy(idx.at[r], idx_v)
        pltpu.async_copy(table.at[idx_v], rows_v, sem).wait()
        pltpu.sync_copy(rows_v, out.at[pl.ds(r * CH, CH)])
        return carry

    lax.fori_loop(0, trip, body, 0)


@functools.cache
def _sc_mesh():
    return plsc.VectorSubcoreMesh(core_axis_name="c", subcore_axis_name="s",
                                  num_cores=NC, num_subcores=NS)


@functools.cache
def _gather16_kernel():
    return pl.kernel(
        _sc_gather_body,
        out_type=jax.ShapeDtypeStruct((E, 16), _f32),
        mesh=_sc_mesh(),
        compiler_params=pltpu.CompilerParams(use_tc_tiling_on_sc=False),
        scratch_types=[
            pltpu.VMEM((CH,), jnp.int32),
            pltpu.VMEM((CH, 16), _f32),
            pltpu.SemaphoreType.DMA,
        ],
    )


def _gather16(table, idx2d):
    return _gather16_kernel()(table, idx2d)


def _sc_scatter16_body(vals, idx, zeros, out, acc, idx2_v, v_v):
    cid, sid, wid, trip = _wid_trip()
    pltpu.sync_copy(zeros.at[pl.ds(sid * SUB_N, SUB_N)],
                    acc.at[pl.ds(sid * SUB_N, SUB_N)])
    plsc.subcore_barrier()

    def body(k, carry):
        r = wid + NW * k
        pltpu.sync_copy(idx.at[r], idx2_v.at[k])
        pltpu.sync_copy(vals.at[pl.ds(r * CH, CH)], v_v)
        pltpu.sync_copy(v_v, acc.at[idx2_v.at[k]], add=True)
        return carry

    lax.fori_loop(0, trip, body, 0)
    plsc.subcore_barrier()
    pltpu.sync_copy(acc.at[pl.ds(sid * SUB_N, SUB_N)],
                    out.at[pl.ds(cid * N + sid * SUB_N, SUB_N)])


@functools.cache
def _scatter16_kernel():
    return pl.kernel(
        _sc_scatter16_body,
        out_type=jax.ShapeDtypeStruct((2 * N, 16), _f32),
        mesh=_sc_mesh(),
        compiler_params=pltpu.CompilerParams(use_tc_tiling_on_sc=False),
        scratch_types=[
            pltpu.VMEM_SHARED((N, 16), _f32),
            pltpu.VMEM((RPW + 1, CH), jnp.int32),
            pltpu.VMEM((CH, 16), _f32),
        ],
    )


def _scatter16(vals, idx2d, zeros):
    return _scatter16_kernel()(vals, idx2d, zeros)


def _sc_gat1_body(elr, rel, src, dst, zeros, ee_out, den_out,
                  acc, idx2_v, src_v, es_v, ed_v, ee_v, sem):
    cid, sid, wid, trip = _wid_trip()
    pltpu.sync_copy(zeros.at[pl.ds(sid * SUB_N, SUB_N)],
                    acc.at[pl.ds(sid * SUB_N, SUB_N)])
    plsc.subcore_barrier()
    mask = lax.iota(jnp.int32, 16) < H

    def chunk(k, carry):
        r = wid + NW * k
        pltpu.sync_copy(src.at[r], src_v)
        pltpu.sync_copy(dst.at[r], idx2_v.at[k])
        pltpu.async_copy(elr.at[src_v], es_v, sem).wait()
        pltpu.async_copy(rel.at[idx2_v.at[k]], ed_v, sem).wait()

        def edge(i, c2):
            x = es_v[i, :] + ed_v[i, :]
            l = jnp.where(x >= 0, x, 0.2 * x)
            ee_v[i, :] = jnp.where(mask, jnp.exp(l), 0.0)
            return c2

        lax.fori_loop(0, CH, edge, 0)
        pltpu.sync_copy(ee_v, ee_out.at[pl.ds(r * CH, CH)])
        pltpu.sync_copy(ee_v, acc.at[idx2_v.at[k]], add=True)
        return carry

    lax.fori_loop(0, trip, chunk, 0)
    plsc.subcore_barrier()
    pltpu.sync_copy(acc.at[pl.ds(sid * SUB_N, SUB_N)],
                    den_out.at[pl.ds(cid * N + sid * SUB_N, SUB_N)])


@functools.cache
def _gat1_kernel():
    return pl.kernel(
        _sc_gat1_body,
        out_type=(jax.ShapeDtypeStruct((E, 16), _f32),
                  jax.ShapeDtypeStruct((2 * N, 16), _f32)),
        mesh=_sc_mesh(),
        compiler_params=pltpu.CompilerParams(use_tc_tiling_on_sc=False),
        scratch_types=[
            pltpu.VMEM_SHARED((N, 16), _f32),
            pltpu.VMEM((RPW + 1, CH), jnp.int32),
            pltpu.VMEM((CH,), jnp.int32),
            pltpu.VMEM((CH, 16), _f32),
            pltpu.VMEM((CH, 16), _f32),
            pltpu.VMEM((CH, 16), _f32),
            pltpu.SemaphoreType.DMA,
        ],
    )


def _gat1(elr, rel, src2d, dst2d, zeros):
    return _gat1_kernel()(elr, rel, src2d, dst2d, zeros)


def _sc_pass2_body(ee, hs, denp, src, dst, dstN, zeros, out,
                   acc, idx2_v, src_v, dn_v, ee_v, hs_v, den_v, den1_v, mu_v,
                   sem):
    cid, sid, wid, trip = _wid_trip()
    pltpu.sync_copy(zeros.at[pl.ds(sid * SUB_N, SUB_N)],
                    acc.at[pl.ds(sid * SUB_N, SUB_N)])
    plsc.subcore_barrier()

    def chunk(k, carry):
        r = wid + NW * k
        pltpu.sync_copy(src.at[r], src_v)
        pltpu.sync_copy(dst.at[r], idx2_v.at[k])
        pltpu.sync_copy(dstN.at[r], dn_v)
        pltpu.sync_copy(ee.at[pl.ds(r * CH, CH)], ee_v)
        pltpu.async_copy(hs.at[src_v], hs_v, sem).wait()
        pltpu.async_copy(denp.at[idx2_v.at[k]], den_v, sem).wait()
        pltpu.async_copy(denp.at[dn_v], den1_v, sem).wait()

        def edge(i, c2):
            hs_row = hs_v[i, :]
            al_row = ee_v[i, :] / (den_v[i, :] + den1_v[i, :])
            for hh in range(H):
                mu_v[i, pl.ds(hh * 16, 16)] = hs_row * al_row[hh]
            return c2

        lax.fori_loop(0, CH, edge, 0)
        pltpu.sync_copy(mu_v, acc.at[idx2_v.at[k]], add=True)
        return carry

    lax.fori_loop(0, trip, chunk, 0)
    plsc.subcore_barrier()
    pltpu.sync_copy(acc.at[pl.ds(sid * SUB_N, SUB_N)],
                    out.at[pl.ds(cid * N + sid * SUB_N, SUB_N)])


@functools.cache
def _pass2_kernel():
    return pl.kernel(
        _sc_pass2_body,
        out_type=jax.ShapeDtypeStruct((2 * N, H * DIN), _f32),
        mesh=_sc_mesh(),
        compiler_params=pltpu.CompilerParams(use_tc_tiling_on_sc=False),
        scratch_types=[
            pltpu.VMEM_SHARED((N, H * DIN), _f32),
            pltpu.VMEM((RPW + 1, CH), jnp.int32),
            pltpu.VMEM((CH,), jnp.int32),
            pltpu.VMEM((CH,), jnp.int32),
            pltpu.VMEM((CH, 16), _f32),
            pltpu.VMEM((CH, 16), _f32),
            pltpu.VMEM((CH, 16), _f32),
            pltpu.VMEM((CH, 16), _f32),
            pltpu.VMEM((CH, H * DIN), _f32),
            pltpu.SemaphoreType.DMA,
        ],
    )


def _pass2(ee, hs, denp, src2d, dst2d, dstN2d, zeros):
    return _pass2_kernel()(ee, hs, denp, src2d, dst2d, dstN2d, zeros)


TE = 4000


def _tc_msg_body(hsrc_ref, e_ref, eW_ref, eb_ref, S_ref, T_ref, out_ref):
    G = jnp.dot(e_ref[...], eW_ref[...],
                preferred_element_type=_f32, precision=lax.Precision.HIGHEST) + eb_ref[...]
    hexp = jnp.dot(hsrc_ref[...], S_ref[...], preferred_element_type=_f32, precision=lax.Precision.HIGHEST)
    out_ref[...] = jnp.dot(G * hexp, T_ref[...], preferred_element_type=_f32, precision=lax.Precision.HIGHEST)


def _msg(hsrc, e, eW, eb, S, T):
    return pl.pallas_call(
        _tc_msg_body,
        grid=(E // TE,),
        in_specs=[
            pl.BlockSpec((TE, 16), lambda i: (i, 0)),
            pl.BlockSpec((TE, 16), lambda i: (i, 0)),
            pl.BlockSpec((16, 256), lambda i: (0, 0)),
            pl.BlockSpec((1, 256), lambda i: (0, 0)),
            pl.BlockSpec((16, 256), lambda i: (0, 0)),
            pl.BlockSpec((256, 16), lambda i: (0, 0)),
        ],
        out_specs=pl.BlockSpec((TE, 16), lambda i: (i, 0)),
        out_shape=jax.ShapeDtypeStruct((E, 16), _f32),
    )(hsrc, e, eW, eb.reshape(1, 256), S, T)


def _bn_core(aggp_ref, cb_ref, g_ref, bb_ref):
    x = aggp_ref[0:N, :] + aggp_ref[N:2 * N, :] + cb_ref[...]
    m = jnp.mean(x, axis=0, keepdims=True)
    xc = x - m
    v = jnp.mean(xc * xc, axis=0, keepdims=True)
    h = g_ref[...] * xc * lax.rsqrt(v + 1e-5) + bb_ref[...]
    return jnp.maximum(h, 0.0)


def _tc_bn_body(aggp_ref, cb_ref, g_ref, bb_ref, out_ref):
    out_ref[...] = _bn_core(aggp_ref, cb_ref, g_ref, bb_ref)


def _bn(aggp, cb, g, bb):
    return pl.pallas_call(
        _tc_bn_body,
        out_shape=jax.ShapeDtypeStruct((N, 16), _f32),
    )(aggp, cb.reshape(1, 16), g.reshape(1, 16), bb.reshape(1, 16))


def _tc_bn4_body(aggp_ref, cb_ref, g_ref, bb_ref, alW_ref, arW_ref,
                 h_ref, elr_ref, rel_ref):
    h = _bn_core(aggp_ref, cb_ref, g_ref, bb_ref)
    h_ref[...] = h
    el = jnp.dot(h, alW_ref[...], preferred_element_type=_f32, precision=lax.Precision.HIGHEST)
    er = jnp.dot(h, arW_ref[...], preferred_element_type=_f32, precision=lax.Precision.HIGHEST)
    elr_ref[...] = jnp.concatenate([el, er], axis=1)
    rel_ref[...] = jnp.concatenate([er, el], axis=1)


def _bn4(aggp, cb, g, bb, alW, arW):
    return pl.pallas_call(
        _tc_bn4_body,
        out_shape=(
            jax.ShapeDtypeStruct((N, 16), _f32),
            jax.ShapeDtypeStruct((N, 16), _f32),
            jax.ShapeDtypeStruct((N, 16), _f32),
        ),
    )(aggp, cb.reshape(1, 16), g.reshape(1, 16), bb.reshape(1, 16), alW, arW)


TCH = 2000


def _tc_final_body(Up_ref, Bb_ref, gbf_ref, gW_ref, gb_ref,
                   f1_ref, b1_ref, f2_ref, b2_ref, f3_ref, b3_ref, out_ref,
                   gate_scr):
    Bb = Bb_ref[...]
    gbf = gbf_ref[...]

    def hf_chunk(c):
        Uc = (Up_ref[pl.ds(c * TCH, TCH), :]
              + Up_ref[pl.ds(N + c * TCH, TCH), :])
        x = jnp.dot(Uc, Bb, preferred_element_type=_f32, precision=lax.Precision.HIGHEST) + gbf
        return jnp.where(x > 0, x, jnp.exp(jnp.minimum(x, 0.0)) - 1.0)

    def p1(c, carry):
        hf = hf_chunk(c)
        gate_scr[pl.ds(c * TCH, TCH), :] = (
            jnp.dot(hf, gW_ref[...], preferred_element_type=_f32, precision=lax.Precision.HIGHEST)
            + gb_ref[...])
        return carry

    lax.fori_loop(0, N // TCH, p1, 0)
    gate = gate_scr[...]
    m = jnp.max(gate)
    eg = jnp.exp(gate - m)
    s = jnp.sum(eg)
    gate_scr[...] = eg

    def p2(c, hg):
        hf = hf_chunk(c)
        w = gate_scr[pl.ds(c * TCH, TCH), :]
        return hg + jnp.sum(w * hf, axis=0, keepdims=True)

    hg = lax.fori_loop(0, N // TCH, p2, jnp.zeros((1, HD), _f32)) / s
    z = jnp.maximum(jnp.dot(hg, f1_ref[...], preferred_element_type=_f32, precision=lax.Precision.HIGHEST)
                    + b1_ref[...], 0.0)
    z = jnp.maximum(jnp.dot(z, f2_ref[...], preferred_element_type=_f32, precision=lax.Precision.HIGHEST)
                    + b2_ref[...], 0.0)
    out_ref[...] = jnp.dot(z, f3_ref[...], preferred_element_type=_f32, precision=lax.Precision.HIGHEST) \
        + b3_ref[...]


def _final(Up, Bblk, gbf, gateW, gateb, fW1, fb1, fW2, fb2, fW3, fb3):
    return pl.pallas_call(
        _tc_final_body,
        out_shape=jax.ShapeDtypeStruct((1, 1), _f32),
        scratch_shapes=[pltpu.VMEM((N, 1), _f32)],
    )(Up, Bblk, gbf, gateW, gateb.reshape(1, 1), fW1, fb1.reshape(1, 64),
      fW2, fb2.reshape(1, 32), fW3, fb3.reshape(1, 1))


def kernel(n, e, edge_index,
           eW1, eb1, cb1, g1, bb1,
           eW2, eb2, cb2, g2, bb2,
           eW3, eb3, cb3, g3, bb3,
           eW4, eb4, cb4, g4, bb4,
           fcW, al, ar, gb, gateW, gateb,
           fW1, fb1, fW2, fb2, fW3, fb3):
    src2d = edge_index[0].reshape(NR, CH)
    dst2d = edge_index[1].reshape(NR, CH)
    dstN2d = dst2d + N
    z16 = jnp.zeros((N, 16), _f32)
    z128 = jnp.zeros((N, H * DIN), _f32)

    fcW3 = fcW.reshape(DIN, H, DOUT)
    alW = jnp.einsum('iho,ho->ih', fcW3, al)
    arW = jnp.einsum('iho,ho->ih', fcW3, ar)
    Bblk = jax.scipy.linalg.block_diag(*[fcW3[:, hh, :] for hh in range(H)])
    eye = jnp.eye(DIN, dtype=_f32)
    S = jnp.kron(eye, jnp.ones((1, 16), _f32))
    T = jnp.tile(eye, (DIN, 1))

    layers = [(eW1, eb1, cb1, g1, bb1), (eW2, eb2, cb2, g2, bb2),
              (eW3, eb3, cb3, g3, bb3), (eW4, eb4, cb4, g4, bb4)]
    h = n
    elr = rel = None
    for li, (eW, eb, cb, g, bb) in enumerate(layers):
        hsrc = _gather16(h, src2d)
        msg = _msg(hsrc, e, eW, eb, S, T)
        aggp = _scatter16(msg, dst2d, z16)
        if li < 3:
            h = _bn(aggp, cb, g, bb)
        else:
            h, elr, rel = _bn4(aggp, cb, g, bb, alW, arW)

    ee16, denp = _gat1(elr, rel, src2d, dst2d, z16)
    Up = _pass2(ee16, h, denp, src2d, dst2d, dstN2d, z128)
    return _final(Up, Bblk, gb.reshape(1, HD), gateW, gateb,
                  fW1, fb1, fW2, fb2, fW3, fb3)
